# baseline (speedup 1.0000x reference)
import functools

import jax
import jax.numpy as jnp
from jax import lax
from jax.experimental import pallas as pl
from jax.experimental.pallas import tpu as pltpu
from jax.experimental.pallas import tpu_sc as plsc

NBINS = 24
NJOINT = NBINS * NBINS
BATCH = 8
NPIX = 256 * 256
NC, NS, LANES = 2, 16, 16
NW = NC * NS
SUB_PER_IMG = NW // BATCH
CHUNK = NPIX // SUB_PER_IMG
SRC = CHUNK * 2
NVEC = CHUNK // LANES


@functools.cache
def _make_sc_hist():
    mesh = plsc.VectorSubcoreMesh(core_axis_name="c", subcore_axis_name="s")
    return functools.partial(
        pl.kernel,
        mesh=mesh,
        out_type=jax.ShapeDtypeStruct((BATCH, SUB_PER_IMG, NJOINT * LANES),
                                      jnp.float32),
        scratch_types=[
            pltpu.VMEM((64, 512), jnp.float32),
            pltpu.VMEM((64, 512), jnp.float32),
            pltpu.VMEM((32,), jnp.float32),
            pltpu.VMEM((NJOINT * LANES,), jnp.float32),
        ],
        compiler_params=pltpu.CompilerParams(needs_layout_passes=False),
    )(_sc_hist_body)


def _sc_hist_body(x_hbm, y_hbm, grid_hbm, out_hbm, xv, yv, grid_v, hist):
    wid = lax.axis_index("s") * NC + lax.axis_index("c")
    img = wid // SUB_PER_IMG
    q = wid % SUB_PER_IMG
    pltpu.sync_copy(x_hbm.at[img, pl.ds(q * 64, 64)], xv)
    pltpu.sync_copy(y_hbm.at[img, pl.ds(q * 64, 64)], yv)
    pltpu.sync_copy(grid_hbm, grid_v)

    zeros = jnp.zeros((LANES,), jnp.float32)

    @plsc.parallel_loop(0, NJOINT, unroll=8)
    def _(j):
        hist[pl.ds(j * LANES, LANES)] = zeros

    lane = lax.iota(jnp.int32, LANES)
    lane2 = lane * 2
    ones = jnp.ones((LANES,), jnp.float32)

    def _bins(v):
        vc = jnp.clip((v + 1.0) * 0.5, 0.001, 0.999)
        b0 = jnp.clip((vc * 24.0).astype(jnp.int32), 0, NBINS - 1)
        g_lo = plsc.load_gather(grid_v, [b0])
        g_hi = plsc.load_gather(grid_v, [b0 + 1])
        return jnp.where(vc <= g_lo, b0 - 1,
                         jnp.where(vc > g_hi, b0 + 1, b0))

    def _loop_body(i, carry):
        row = jnp.broadcast_to(i >> 4, (LANES,)).astype(jnp.int32)
        col = (i & 15) * 32 + lane2
        gx = plsc.load_gather(xv, [row, col])
        gy = plsc.load_gather(yv, [row, col])
        xb = _bins(gx)
        yb = _bins(gy)
        addr = (xb * NBINS + yb) * LANES + lane
        plsc.addupdate_scatter(hist, [addr], ones)
        return carry

    lax.fori_loop(0, NVEC, _loop_body, 0, unroll=8)

    pltpu.sync_copy(hist, out_hbm.at[img, q])


def _tc_nmi_body(h_ref, o_ref):
    h = h_ref[...]
    c = jnp.sum(jnp.sum(h, axis=3), axis=1)
    total = jnp.sum(c, axis=1, keepdims=True) + 1e-10
    p = c / total

    k = lax.broadcasted_iota(jnp.int32, (NJOINT, NBINS), 0)
    i = lax.broadcasted_iota(jnp.int32, (NJOINT, NBINS), 1)
    row_ind = (k // NBINS == i).astype(jnp.float32)
    col_ind = (k % NBINS == i).astype(jnp.float32)
    xh = jnp.dot(p, row_ind, preferred_element_type=jnp.float32,
                 precision=lax.Precision.HIGHEST)
    yh = jnp.dot(p, col_ind, preferred_element_type=jnp.float32,
                 precision=lax.Precision.HIGHEST)

    eps = 1e-5
    jh = p + eps
    lx = jnp.log(xh + eps)
    ly = jnp.log(yh + eps)
    t1 = jnp.sum(jh * jnp.log(jh), axis=1)
    t2 = jnp.sum((xh + NBINS * eps) * lx, axis=1)
    t3 = jnp.sum((yh + NBINS * eps) * ly, axis=1)
    mi = t1 - t2 - t3
    ent = -jnp.sum((xh + eps) * lx, axis=1) - jnp.sum((yh + eps) * ly, axis=1)
    nmi = jnp.where(ent < 1e-10, 0.0, 2.0 * mi / ent)
    nmi = jnp.clip(nmi, -1.0, 1.0)
    m = jnp.sum(nmi) / BATCH
    o_ref[0, 0] = -jnp.clip(m, -1.0, 1.0)


def kernel(x, y):
    with jax.ensure_compile_time_eval():
        nkey = jax.random.key(1)
        kx, ky = jax.random.split(nkey)
        nx = jax.random.normal(kx, (BATCH, 1, 256, 256), jnp.float32) * 0.0001
        ny = jax.random.normal(ky, (BATCH, 1, 256, 256), jnp.float32) * 0.0001
        nxw = jnp.zeros((BATCH, 256, 512), jnp.float32)
        nxw = nxw.at[:, :, ::2].set(nx[:, 0])
        nyw = jnp.zeros((BATCH, 256, 512), jnp.float32)
        nyw = nyw.at[:, :, ::2].set(ny[:, 0])
        grid = jnp.concatenate([jnp.linspace(0.0, 1.0, NBINS + 1),
                                jnp.full((7,), 2.0, jnp.float32)])
    xr = x[:, 0, ::2, :] + nxw
    yr = y[:, 0, ::2, :] + nyw
    hist = _make_sc_hist()(xr, yr, grid)
    h4 = hist.reshape(BATCH, SUB_PER_IMG, NJOINT, LANES)
    out = pl.pallas_call(
        _tc_nmi_body,
        out_shape=jax.ShapeDtypeStruct((1, 1), jnp.float32),
        out_specs=pl.BlockSpec(memory_space=pltpu.SMEM),
    )(h4)
    return out.reshape(())


# speedup vs baseline: 1.7376x; 1.7376x over previous
import functools

import jax
import jax.numpy as jnp
from jax import lax
from jax.experimental import pallas as pl
from jax.experimental.pallas import tpu as pltpu
from jax.experimental.pallas import tpu_sc as plsc

NBINS = 24
NJOINT = NBINS * NBINS
BATCH = 8
NPIX = 256 * 256
NC, NS, LANES = 2, 16, 16
NW = NC * NS
SUB_PER_IMG = NW // BATCH
CHUNK = NPIX // SUB_PER_IMG
SRC = CHUNK * 2
NVEC = CHUNK // LANES


@functools.cache
def _make_sc_hist():
    mesh = plsc.VectorSubcoreMesh(core_axis_name="c", subcore_axis_name="s")
    return functools.partial(
        pl.kernel,
        mesh=mesh,
        out_type=jax.ShapeDtypeStruct((BATCH, SUB_PER_IMG, NJOINT * LANES),
                                      jnp.float32),
        scratch_types=[
            pltpu.VMEM((64, 512), jnp.float32),
            pltpu.VMEM((64, 512), jnp.float32),
            pltpu.VMEM((32,), jnp.float32),
            pltpu.VMEM((NJOINT * LANES,), jnp.float32),
        ],
        compiler_params=pltpu.CompilerParams(needs_layout_passes=False),
    )(_sc_hist_body)


def _sc_hist_body(x_hbm, y_hbm, grid_hbm, out_hbm, xv, yv, grid_v, hist):
    wid = lax.axis_index("s") * NC + lax.axis_index("c")
    img = wid // SUB_PER_IMG
    q = wid % SUB_PER_IMG
    pltpu.sync_copy(grid_hbm, grid_v)

    zeros = jnp.zeros((LANES,), jnp.float32)

    @plsc.parallel_loop(0, NJOINT, unroll=8)
    def _(j):
        hist[pl.ds(j * LANES, LANES)] = zeros

    lane = lax.iota(jnp.int32, LANES)
    lane2 = lane * 2
    ones = jnp.ones((LANES,), jnp.float32)

    def _bins(v):
        vc = jnp.clip((v + 1.0) * 0.5, 0.001, 0.999)
        b0 = jnp.clip((vc * 24.0).astype(jnp.int32), 0, NBINS - 1)
        g_lo = plsc.load_gather(grid_v, [b0])
        g_hi = plsc.load_gather(grid_v, [b0 + 1])
        return jnp.where(vc <= g_lo, b0 - 1,
                         jnp.where(vc > g_hi, b0 + 1, b0))

    def body(i):
        row = jnp.broadcast_to((i >> 4) * 2, (LANES,)).astype(jnp.int32)
        col = (i & 15) * 32 + lane2
        gx = plsc.load_gather(xv, [row, col])
        gy = plsc.load_gather(yv, [row, col])
        xb = _bins(gx)
        yb = _bins(gy)
        addr = (xb * NBINS + yb) * LANES + lane
        plsc.addupdate_scatter(hist, [addr], ones)

    for h in range(2):
        r0 = q * 128 + h * 64
        pltpu.sync_copy(x_hbm.at[img, pl.ds(r0, 64)], xv)
        pltpu.sync_copy(y_hbm.at[img, pl.ds(r0, 64)], yv)
        plsc.parallel_loop(0, NVEC // 2, unroll=8)(body)

    pltpu.sync_copy(hist, out_hbm.at[img, q])


def _tc_nmi_body(h_ref, o_ref):
    h = h_ref[...]
    c = jnp.sum(jnp.sum(h, axis=3), axis=1)
    total = jnp.sum(c, axis=1, keepdims=True) + 1e-10
    p = c / total

    k = lax.broadcasted_iota(jnp.int32, (NJOINT, NBINS), 0)
    i = lax.broadcasted_iota(jnp.int32, (NJOINT, NBINS), 1)
    row_ind = (k // NBINS == i).astype(jnp.float32)
    col_ind = (k % NBINS == i).astype(jnp.float32)
    xh = jnp.dot(p, row_ind, preferred_element_type=jnp.float32,
                 precision=lax.Precision.HIGHEST)
    yh = jnp.dot(p, col_ind, preferred_element_type=jnp.float32,
                 precision=lax.Precision.HIGHEST)

    eps = 1e-5
    jh = p + eps
    lx = jnp.log(xh + eps)
    ly = jnp.log(yh + eps)
    t1 = jnp.sum(jh * jnp.log(jh), axis=1)
    t2 = jnp.sum((xh + NBINS * eps) * lx, axis=1)
    t3 = jnp.sum((yh + NBINS * eps) * ly, axis=1)
    mi = t1 - t2 - t3
    ent = -jnp.sum((xh + eps) * lx, axis=1) - jnp.sum((yh + eps) * ly, axis=1)
    nmi = jnp.where(ent < 1e-10, 0.0, 2.0 * mi / ent)
    nmi = jnp.clip(nmi, -1.0, 1.0)
    m = jnp.sum(nmi) / BATCH
    o_ref[0, 0] = -jnp.clip(m, -1.0, 1.0)


def kernel(x, y):
    with jax.ensure_compile_time_eval():
        nkey = jax.random.key(1)
        kx, ky = jax.random.split(nkey)
        nx = jax.random.normal(kx, (BATCH, 1, 256, 256), jnp.float32) * 0.0001
        ny = jax.random.normal(ky, (BATCH, 1, 256, 256), jnp.float32) * 0.0001
        nxw = jnp.zeros((BATCH, 512, 512), jnp.float32)
        nxw = nxw.at[:, ::2, ::2].set(nx[:, 0])
        nyw = jnp.zeros((BATCH, 512, 512), jnp.float32)
        nyw = nyw.at[:, ::2, ::2].set(ny[:, 0])
        grid = jnp.concatenate([jnp.linspace(0.0, 1.0, NBINS + 1),
                                jnp.full((7,), 2.0, jnp.float32)])
    xr = x[:, 0] + nxw
    yr = y[:, 0] + nyw
    hist = _make_sc_hist()(xr, yr, grid)
    h4 = hist.reshape(BATCH, SUB_PER_IMG, NJOINT, LANES)
    out = pl.pallas_call(
        _tc_nmi_body,
        out_shape=jax.ShapeDtypeStruct((1, 1), jnp.float32),
        out_specs=pl.BlockSpec(memory_space=pltpu.SMEM),
    )(h4)
    return out.reshape(())


# speedup vs baseline: 1.9472x; 1.1207x over previous
import functools

import jax
import jax.numpy as jnp
from jax import lax
from jax.experimental import pallas as pl
from jax.experimental.pallas import tpu as pltpu
from jax.experimental.pallas import tpu_sc as plsc

NBINS = 24
NJOINT = NBINS * NBINS
BATCH = 8
NPIX = 256 * 256
NC, NS, LANES = 2, 16, 16
NW = NC * NS
SUB_PER_IMG = NW // BATCH
CHUNK = NPIX // SUB_PER_IMG
SRC = CHUNK * 2
NVEC = CHUNK // LANES


@functools.cache
def _make_sc_hist():
    mesh = plsc.VectorSubcoreMesh(core_axis_name="c", subcore_axis_name="s")
    return functools.partial(
        pl.kernel,
        mesh=mesh,
        out_type=jax.ShapeDtypeStruct((BATCH, SUB_PER_IMG, NJOINT * LANES),
                                      jnp.float32),
        scratch_types=[
            pltpu.VMEM((64, 512), jnp.float32),
            pltpu.VMEM((64, 512), jnp.float32),
            pltpu.VMEM((32,), jnp.float32),
            pltpu.VMEM((NJOINT * LANES,), jnp.float32),
        ],
        compiler_params=pltpu.CompilerParams(needs_layout_passes=False),
    )(_sc_hist_body)


def _sc_hist_body(x_hbm, y_hbm, grid_hbm, out_hbm, xv, yv, grid_v, hist):
    wid = lax.axis_index("s") * NC + lax.axis_index("c")
    img = wid // SUB_PER_IMG
    q = wid % SUB_PER_IMG
    pltpu.sync_copy(grid_hbm, grid_v)

    zeros = jnp.zeros((LANES,), jnp.float32)

    @plsc.parallel_loop(0, NJOINT, unroll=8)
    def _(j):
        hist[pl.ds(j * LANES, LANES)] = zeros

    lane = lax.iota(jnp.int32, LANES)
    lane2 = lane * 2
    ones = jnp.ones((LANES,), jnp.float32)

    def _bins(v):
        vc = jnp.clip((v + 1.0) * 0.5, 0.001, 0.999)
        b0 = jnp.clip((vc * 24.0).astype(jnp.int32), 0, NBINS - 1)
        g_lo = plsc.load_gather(grid_v, [b0])
        g_hi = plsc.load_gather(grid_v, [b0 + 1])
        return jnp.where(vc <= g_lo, b0 - 1,
                         jnp.where(vc > g_hi, b0 + 1, b0))

    def body(i):
        row = jnp.broadcast_to((i >> 4) * 2, (LANES,)).astype(jnp.int32)
        col = (i & 15) * 32 + lane2
        gx = plsc.load_gather(xv, [row, col])
        gy = plsc.load_gather(yv, [row, col])
        xb = _bins(gx)
        yb = _bins(gy)
        addr = (xb * NBINS + yb) * LANES + lane
        plsc.addupdate_scatter(hist, [addr], ones)

    for h in range(2):
        r0 = q * 128 + h * 64
        pltpu.sync_copy(x_hbm.at[img, pl.ds(r0, 64)], xv)
        pltpu.sync_copy(y_hbm.at[img, pl.ds(r0, 64)], yv)
        plsc.parallel_loop(0, NVEC // 2, unroll=8)(body)

    pltpu.sync_copy(hist, out_hbm.at[img, q])


def _tc_nmi_body(h_ref, o_ref):
    h = h_ref[...].reshape(BATCH, SUB_PER_IMG, NJOINT, LANES)
    c = jnp.sum(jnp.sum(h, axis=3), axis=1)
    total = jnp.sum(c, axis=1, keepdims=True) + 1e-10
    p = c / total

    k = lax.broadcasted_iota(jnp.int32, (NJOINT, NBINS), 0)
    i = lax.broadcasted_iota(jnp.int32, (NJOINT, NBINS), 1)
    row_ind = (k // NBINS == i).astype(jnp.float32)
    col_ind = (k % NBINS == i).astype(jnp.float32)
    xh = jnp.dot(p, row_ind, preferred_element_type=jnp.float32,
                 precision=lax.Precision.HIGHEST)
    yh = jnp.dot(p, col_ind, preferred_element_type=jnp.float32,
                 precision=lax.Precision.HIGHEST)

    eps = 1e-5
    jh = p + eps
    lx = jnp.log(xh + eps)
    ly = jnp.log(yh + eps)
    t1 = jnp.sum(jh * jnp.log(jh), axis=1)
    t2 = jnp.sum((xh + NBINS * eps) * lx, axis=1)
    t3 = jnp.sum((yh + NBINS * eps) * ly, axis=1)
    mi = t1 - t2 - t3
    ent = -jnp.sum((xh + eps) * lx, axis=1) - jnp.sum((yh + eps) * ly, axis=1)
    nmi = jnp.where(ent < 1e-10, 0.0, 2.0 * mi / ent)
    nmi = jnp.clip(nmi, -1.0, 1.0)
    m = jnp.sum(nmi) / BATCH
    o_ref[0, 0] = -jnp.clip(m, -1.0, 1.0)


def kernel(x, y):
    with jax.ensure_compile_time_eval():
        nkey = jax.random.key(1)
        kx, ky = jax.random.split(nkey)
        nx = jax.random.normal(kx, (BATCH, 1, 256, 256), jnp.float32) * 0.0001
        ny = jax.random.normal(ky, (BATCH, 1, 256, 256), jnp.float32) * 0.0001
        nxw = jnp.zeros((BATCH, 512, 512), jnp.float32)
        nxw = nxw.at[:, ::2, ::2].set(nx[:, 0])
        nyw = jnp.zeros((BATCH, 512, 512), jnp.float32)
        nyw = nyw.at[:, ::2, ::2].set(ny[:, 0])
        grid = jnp.concatenate([jnp.linspace(0.0, 1.0, NBINS + 1),
                                jnp.full((7,), 2.0, jnp.float32)])
    xr = x[:, 0] + nxw
    yr = y[:, 0] + nyw
    hist = _make_sc_hist()(xr, yr, grid)
    out = pl.pallas_call(
        _tc_nmi_body,
        out_shape=jax.ShapeDtypeStruct((1, 1), jnp.float32),
        out_specs=pl.BlockSpec(memory_space=pltpu.SMEM),
    )(hist)
    return out.reshape(())


# speedup vs baseline: 2.2545x; 1.1578x over previous
import functools

import jax
import jax.numpy as jnp
from jax import lax
from jax.experimental import pallas as pl
from jax.experimental.pallas import tpu as pltpu
from jax.experimental.pallas import tpu_sc as plsc

NBINS = 24
NJOINT = NBINS * NBINS
BATCH = 8
NPIX = 256 * 256
NC, NS, LANES = 2, 16, 16
NW = NC * NS
SUB_PER_IMG = NW // BATCH
CHUNK = NPIX // SUB_PER_IMG
SRC = CHUNK * 2
NVEC = CHUNK // LANES


@functools.cache
def _make_sc_hist():
    mesh = plsc.VectorSubcoreMesh(core_axis_name="c", subcore_axis_name="s")
    return functools.partial(
        pl.kernel,
        mesh=mesh,
        out_type=jax.ShapeDtypeStruct((BATCH, SUB_PER_IMG, NJOINT * LANES),
                                      jnp.float32),
        scratch_types=[
            pltpu.VMEM((64, 512), jnp.float32),
            pltpu.VMEM((64, 512), jnp.float32),
            pltpu.VMEM((CHUNK // 2,), jnp.float32),
            pltpu.VMEM((CHUNK // 2,), jnp.float32),
            pltpu.VMEM((32,), jnp.float32),
            pltpu.VMEM((NJOINT * LANES,), jnp.float32),
        ],
        compiler_params=pltpu.CompilerParams(needs_layout_passes=False),
    )(_sc_hist_body)


def _sc_hist_body(x_hbm, y_hbm, nx_hbm, ny_hbm, grid_hbm, out_hbm,
                  xv, yv, nxv, nyv, grid_v, hist):
    wid = lax.axis_index("s") * NC + lax.axis_index("c")
    img = wid // SUB_PER_IMG
    q = wid % SUB_PER_IMG
    pltpu.sync_copy(grid_hbm, grid_v)

    zeros = jnp.zeros((LANES,), jnp.float32)

    @plsc.parallel_loop(0, NJOINT, unroll=8)
    def _(j):
        hist[pl.ds(j * LANES, LANES)] = zeros

    lane = lax.iota(jnp.int32, LANES)
    lane2 = lane * 2
    ones = jnp.ones((LANES,), jnp.float32)

    def _bins(v):
        vc = jnp.clip((v + 1.0) * 0.5, 0.001, 0.999)
        b0 = jnp.clip((vc * 24.0).astype(jnp.int32), 0, NBINS - 1)
        g_lo = plsc.load_gather(grid_v, [b0])
        g_hi = plsc.load_gather(grid_v, [b0 + 1])
        return jnp.where(vc <= g_lo, b0 - 1,
                         jnp.where(vc > g_hi, b0 + 1, b0))

    def body(i):
        row = jnp.broadcast_to((i >> 4) * 2, (LANES,)).astype(jnp.int32)
        col = (i & 15) * 32 + lane2
        gx = plsc.load_gather(xv, [row, col])
        gy = plsc.load_gather(yv, [row, col])
        xb = _bins(gx + nxv[pl.ds(i * LANES, LANES)])
        yb = _bins(gy + nyv[pl.ds(i * LANES, LANES)])
        addr = (xb * NBINS + yb) * LANES + lane
        plsc.addupdate_scatter(hist, [addr], ones)

    for h in range(2):
        r0 = q * 128 + h * 64
        n0 = wid * CHUNK + h * (CHUNK // 2)
        pltpu.sync_copy(x_hbm.at[img, pl.ds(r0, 64)], xv)
        pltpu.sync_copy(y_hbm.at[img, pl.ds(r0, 64)], yv)
        pltpu.sync_copy(nx_hbm.at[pl.ds(n0, CHUNK // 2)], nxv)
        pltpu.sync_copy(ny_hbm.at[pl.ds(n0, CHUNK // 2)], nyv)
        plsc.parallel_loop(0, NVEC // 2, unroll=8)(body)

    pltpu.sync_copy(hist, out_hbm.at[img, q])


def _tc_nmi_body(h_ref, o_ref):
    h = h_ref[...].reshape(BATCH, SUB_PER_IMG, NJOINT, LANES)
    c = jnp.sum(jnp.sum(h, axis=3), axis=1)
    total = jnp.sum(c, axis=1, keepdims=True) + 1e-10
    p = c / total

    k = lax.broadcasted_iota(jnp.int32, (NJOINT, NBINS), 0)
    i = lax.broadcasted_iota(jnp.int32, (NJOINT, NBINS), 1)
    row_ind = (k // NBINS == i).astype(jnp.float32)
    col_ind = (k % NBINS == i).astype(jnp.float32)
    xh = jnp.dot(p, row_ind, preferred_element_type=jnp.float32,
                 precision=lax.Precision.HIGHEST)
    yh = jnp.dot(p, col_ind, preferred_element_type=jnp.float32,
                 precision=lax.Precision.HIGHEST)

    eps = 1e-5
    jh = p + eps
    lx = jnp.log(xh + eps)
    ly = jnp.log(yh + eps)
    t1 = jnp.sum(jh * jnp.log(jh), axis=1)
    t2 = jnp.sum((xh + NBINS * eps) * lx, axis=1)
    t3 = jnp.sum((yh + NBINS * eps) * ly, axis=1)
    mi = t1 - t2 - t3
    ent = -jnp.sum((xh + eps) * lx, axis=1) - jnp.sum((yh + eps) * ly, axis=1)
    nmi = jnp.where(ent < 1e-10, 0.0, 2.0 * mi / ent)
    nmi = jnp.clip(nmi, -1.0, 1.0)
    m = jnp.sum(nmi) / BATCH
    o_ref[0, 0] = -jnp.clip(m, -1.0, 1.0)


def kernel(x, y):
    with jax.ensure_compile_time_eval():
        nkey = jax.random.key(1)
        kx, ky = jax.random.split(nkey)
        nx = jax.random.normal(kx, (BATCH, 1, 256, 256), jnp.float32) * 0.0001
        ny = jax.random.normal(ky, (BATCH, 1, 256, 256), jnp.float32) * 0.0001
        nxf = nx.reshape(-1)
        nyf = ny.reshape(-1)
        grid = jnp.concatenate([jnp.linspace(0.0, 1.0, NBINS + 1),
                                jnp.full((7,), 2.0, jnp.float32)])
    xr = x[:, 0]
    yr = y[:, 0]
    hist = _make_sc_hist()(xr, yr, nxf, nyf, grid)
    out = pl.pallas_call(
        _tc_nmi_body,
        out_shape=jax.ShapeDtypeStruct((1, 1), jnp.float32),
        out_specs=pl.BlockSpec(memory_space=pltpu.SMEM),
    )(hist)
    return out.reshape(())
